# Initial kernel scaffold; baseline (speedup 1.0000x reference)
import jax
import jax.numpy as jnp
from jax.experimental import pallas as pl


def kernel(queries, keys, values, attn_mask, position_embedding_key, d_keys, u_W, u_b, v_W, v_b):
    raise NotImplementedError("write your pallas kernel here")


# speedup vs baseline: 5.3472x; 5.3472x over previous
from functools import partial

import numpy as np
import jax
import jax.numpy as jnp
from jax import lax
from jax.experimental import pallas as pl
from jax.experimental.pallas import tpu as pltpu

B, L, H, D = 2, 2048, 12, 64
U = 40
SCALE = 0.125
NEG = -1e30


def _build_cnt_t() -> np.ndarray:
    cpu = jax.devices("cpu")[0]
    with jax.default_device(cpu):
        idx = np.asarray(jax.random.randint(jax.random.key(42), (L, U), 0, L))
    cnt_t = np.zeros((L, L), np.int8)
    np.add.at(cnt_t, (idx, np.broadcast_to(np.arange(L)[:, None], (L, U))), 1)
    return cnt_t


_CNT_T = _build_cnt_t()


def _body(q_ref, k_ref, v_ref, pek_ref, cnt_ref, uw_ref, vw_ref, ub_ref,
          vb_ref, out_ref, oh_ref):
    Q = q_ref[0, 0, :, :]
    K = k_ref[0, 0, :, :]
    V = v_ref[0, 0, :, :]
    PEK = pek_ref[0, 0, :, :]

    f32 = jnp.float32
    dot = partial(lax.dot_general, preferred_element_type=f32)

    St = dot(K, Q, (((1,), (1,)), ((), ())))
    cnt = cnt_ref[...].astype(f32)
    smax = jnp.max(jnp.where(cnt > 0.0, St, NEG), axis=0, keepdims=True)
    ssum = jnp.sum(St * cnt, axis=0, keepdims=True)
    M0 = smax - ssum * (1.0 / L)

    iota = lax.broadcasted_iota(jnp.int32, (1, L), 1)

    def topk_body(i, Mv):
        maxv = jnp.max(Mv)
        idx = jnp.min(jnp.where(Mv == maxv, iota, L))
        oh_ref[pl.ds(i, 1), :] = (iota == idx).astype(f32)
        return jnp.where(iota == idx, NEG, Mv)

    lax.fori_loop(0, U, topk_body, M0)
    OH = oh_ref[...]

    Qr = dot(OH, Q, (((1,), (0,)), ((), ())))
    KP = K + PEK
    G = dot(KP, Qr, (((1,), (1,)), ((), ())))
    G = G + dot(K, uw_ref[...], (((1,), (1,)), ((), ())))
    G = G + dot(PEK, vw_ref[...], (((1,), (1,)), ((), ())))
    G = (G + (ub_ref[...] + vb_ref[...])) * SCALE

    colmax = jnp.max(G, axis=0, keepdims=True)
    E = jnp.exp(G - colmax)
    A = E / jnp.sum(E, axis=0, keepdims=True)
    upd = dot(A, V, (((0,), (0,)), ((), ())))

    vmean = jnp.mean(V, axis=0, keepdims=True)
    outb = jnp.broadcast_to(vmean, (L, D)) + dot(
        OH, upd - vmean, (((0,), (0,)), ((), ())))
    out_ref[0, 0, :, :] = outb


def kernel(queries, keys, values, attn_mask, position_embedding_key, d_keys,
           u_W, u_b, v_W, v_b):
    del attn_mask, d_keys
    cnt_t = jnp.asarray(_CNT_T)
    tr = lambda x: jnp.transpose(x, (0, 2, 1, 3))
    big = pl.BlockSpec((1, 1, L, D), lambda b, h: (b, h, 0, 0))
    const2 = lambda shape: pl.BlockSpec(shape, lambda b, h: (0, 0))
    out = pl.pallas_call(
        _body,
        grid=(B, H),
        in_specs=[big, big, big, big,
                  const2((L, L)), const2((U, D)), const2((U, D)),
                  const2((1, U)), const2((1, U))],
        out_specs=big,
        out_shape=jax.ShapeDtypeStruct((B, H, L, D), jnp.float32),
        scratch_shapes=[pltpu.VMEM((U, L), jnp.float32)],
    )(tr(queries), tr(keys), tr(values), tr(position_embedding_key), cnt_t,
      u_W, v_W, u_b.reshape(1, U), v_b.reshape(1, U))
    return jnp.transpose(out, (0, 2, 1, 3))


# speedup vs baseline: 6.2805x; 1.1745x over previous
from functools import partial

import numpy as np
import jax
import jax.numpy as jnp
from jax import lax
from jax.experimental import pallas as pl
from jax.experimental.pallas import tpu as pltpu

B, L, H, D = 2, 2048, 12, 64
U = 40
SCALE = 0.125
NEG = -1e30
HPS = 2


def _threefry2x32(k1, k2, x0, x1):
    def rotl(x, d):
        return ((x << np.uint32(d)) | (x >> np.uint32(32 - d))).astype(np.uint32)
    x0 = x0.astype(np.uint32).copy()
    x1 = x1.astype(np.uint32).copy()
    ks = [np.uint32(k1), np.uint32(k2),
          np.uint32(np.uint32(k1) ^ np.uint32(k2) ^ np.uint32(0x1BD11BDA))]
    R = [(13, 15, 26, 6), (17, 29, 16, 24)]
    x0 = (x0 + ks[0]).astype(np.uint32)
    x1 = (x1 + ks[1]).astype(np.uint32)
    for i in range(5):
        for r in R[i % 2]:
            x0 = (x0 + x1).astype(np.uint32)
            x1 = rotl(x1, r) ^ x0
        x0 = (x0 + ks[(i + 1) % 3]).astype(np.uint32)
        x1 = (x1 + ks[(i + 2) % 3] + np.uint32(i + 1)).astype(np.uint32)
    return x0, x1


def _build_cnt_t() -> np.ndarray:
    b1, b2 = _threefry2x32(0, 42, np.zeros(2, np.uint32),
                           np.arange(2, dtype=np.uint32))
    lo1, lo2 = _threefry2x32(b1[1], b2[1], np.zeros(L * U, np.uint32),
                             np.arange(L * U, dtype=np.uint32))
    idx = ((lo1 ^ lo2) % np.uint32(L)).astype(np.int64).reshape(L, U)
    cnt_t = np.zeros((L, L), np.int8)
    np.add.at(cnt_t, (idx, np.broadcast_to(np.arange(L)[:, None], (L, U))), 1)
    return cnt_t


_CNT_T = _build_cnt_t()


def _body(q_ref, k_ref, v_ref, pek_ref, cnt_ref, uw_ref, vw_ref, ub_ref,
          vb_ref, out_ref, oh_ref):
    f32 = jnp.float32
    dot = partial(lax.dot_general, preferred_element_type=f32)
    cnt = cnt_ref[...].astype(f32)

    Ms = []
    for hh in range(HPS):
        sl = slice(hh * D, (hh + 1) * D)
        St = dot(k_ref[:, sl], q_ref[:, sl], (((1,), (1,)), ((), ())))
        smax = jnp.max(jnp.where(cnt > 0.0, St, NEG), axis=0, keepdims=True)
        ssum = jnp.sum(St * cnt, axis=0, keepdims=True)
        Ms.append(smax - ssum * (1.0 / L))
    M0 = jnp.concatenate(Ms, axis=0)

    iota = lax.broadcasted_iota(jnp.int32, (HPS, L), 1)

    def topk_body(i, Mv):
        maxv = jnp.max(Mv, axis=1, keepdims=True)
        idx = jnp.min(jnp.where(Mv == maxv, iota, L), axis=1, keepdims=True)
        hit = iota == idx
        oh_ref[:, pl.ds(i, 1), :] = hit.astype(f32)[:, None, :]
        return jnp.where(hit, NEG, Mv)

    lax.fori_loop(0, U, topk_body, M0)

    bias = ub_ref[...] + vb_ref[...]
    for hh in range(HPS):
        sl = slice(hh * D, (hh + 1) * D)
        Q = q_ref[:, sl]
        K = k_ref[:, sl]
        V = v_ref[:, sl]
        PEK = pek_ref[:, sl]
        OH = oh_ref[hh]
        Qr = dot(OH, Q, (((1,), (0,)), ((), ())))
        G = dot(K + PEK, Qr, (((1,), (1,)), ((), ())))
        G = G + dot(K, uw_ref[...], (((1,), (1,)), ((), ())))
        G = G + dot(PEK, vw_ref[...], (((1,), (1,)), ((), ())))
        G = (G + bias) * SCALE
        colmax = jnp.max(G, axis=0, keepdims=True)
        E = jnp.exp(G - colmax)
        A = E / jnp.sum(E, axis=0, keepdims=True)
        upd = dot(A, V, (((0,), (0,)), ((), ())))
        vmean = jnp.mean(V, axis=0, keepdims=True)
        out_ref[:, sl] = jnp.broadcast_to(vmean, (L, D)) + dot(
            OH, upd - vmean, (((0,), (0,)), ((), ())))


def kernel(queries, keys, values, attn_mask, position_embedding_key, d_keys,
           u_W, u_b, v_W, v_b):
    del attn_mask, d_keys
    cnt_t = jnp.asarray(_CNT_T)
    rs = lambda x: x.reshape(B * L, H * D)
    big = pl.BlockSpec((L, HPS * D), lambda b, h2: (b, h2))
    const2 = lambda shape: pl.BlockSpec(shape, lambda b, h2: (0, 0))
    out = pl.pallas_call(
        _body,
        grid=(B, H // HPS),
        in_specs=[big, big, big, big,
                  const2((L, L)), const2((U, D)), const2((U, D)),
                  const2((1, U)), const2((1, U))],
        out_specs=big,
        out_shape=jax.ShapeDtypeStruct((B * L, H * D), jnp.float32),
        scratch_shapes=[pltpu.VMEM((HPS, U, L), jnp.float32)],
    )(rs(queries), rs(keys), rs(values), rs(position_embedding_key), cnt_t,
      u_W, v_W, u_b.reshape(1, U), v_b.reshape(1, U))
    return out.reshape(B, L, H, D)
